# baseline (speedup 1.0000x reference)
import functools
import jax
import jax.numpy as jnp
from jax import lax
from jax.experimental import pallas as pl

S_SPLIT = 812
K_TOP = 3
NEG_BIG = -1e30


def _transform_body(xh_ref, xt_ref, wh_ref, bh_ref, wt_ref, bt_ref,
                    oh_ref, ot_ref):
    xh = xh_ref[...]
    xt = xt_ref[...]
    oh_ref[...] = jnp.dot(xh, wh_ref[...],
                          preferred_element_type=jnp.float32) + bh_ref[...]
    ot_ref[...] = jnp.dot(xt, wt_ref[...],
                          preferred_element_type=jnp.float32) + bt_ref[...]


def _transforms(x_homo, x_hete, W_homo, b_homo, W_hete, b_hete):
    N, D = x_homo.shape
    O = W_homo.shape[0]
    BLK = 1000
    grid = (N // BLK,)
    whT = W_homo.T.reshape(D, O)
    wtT = W_hete.T.reshape(D, O)
    bh = b_homo.reshape(1, O)
    bt = b_hete.reshape(1, O)
    out = pl.pallas_call(
        _transform_body,
        grid=grid,
        in_specs=[
            pl.BlockSpec((BLK, D), lambda i: (i, 0)),
            pl.BlockSpec((BLK, D), lambda i: (i, 0)),
            pl.BlockSpec((D, O), lambda i: (0, 0)),
            pl.BlockSpec((1, O), lambda i: (0, 0)),
            pl.BlockSpec((D, O), lambda i: (0, 0)),
            pl.BlockSpec((1, O), lambda i: (0, 0)),
        ],
        out_specs=[
            pl.BlockSpec((BLK, O), lambda i: (i, 0)),
            pl.BlockSpec((BLK, O), lambda i: (i, 0)),
        ],
        out_shape=[
            jax.ShapeDtypeStruct((N, O), jnp.float32),
            jax.ShapeDtypeStruct((N, O), jnp.float32),
        ],
    )(x_homo, x_hete, whT, bh, wtT, bt)
    return out


def _topk_bilinear_body(ncols, d_ref, h_ref, f_ref, bmat_ref, bbil_ref,
                        xh_ref, xt_ref, gate_ref, out_ref):
    D = d_ref[...]
    R, C = D.shape
    col = lax.broadcasted_iota(jnp.int32, (R, C), 1)
    D = jnp.where(col < ncols, D, NEG_BIG)
    W = jnp.zeros((R, C), jnp.float32)
    cnt = jnp.zeros((R, 1), jnp.float32)
    for _ in range(K_TOP):
        m = jnp.max(D, axis=1, keepdims=True)
        is_max = D == m
        idx = jnp.min(jnp.where(is_max, col, jnp.int32(2**30)),
                      axis=1, keepdims=True)
        sel = col == idx
        valid = m > 0.0
        W = W + jnp.where(sel & valid, 1.0, 0.0)
        cnt = cnt + jnp.where(valid, 1.0, 0.0)
        D = jnp.where(sel, NEG_BIG, D)
    g = jnp.dot(W, h_ref[...], preferred_element_type=jnp.float32)
    g = g / jnp.maximum(cnt, 1.0)
    f = f_ref[...]
    O = f.shape[1]
    P = (f[:, :, None] * g[:, None, :]).reshape(R, O * O)
    mean = jnp.dot(P, bmat_ref[...],
                   preferred_element_type=jnp.float32) + bbil_ref[...]
    mean = jnp.where(cnt > 0.0, mean, 0.0)
    gate = jax.nn.sigmoid(gate_ref[...])
    leak = jnp.where(mean >= 0.0, mean, 0.01 * mean)
    fused = gate * leak + (1.0 - gate) * xt_ref[...]
    out_ref[...] = xh_ref[...] + fused


def _topk_bilinear(Dpad, H, f, xh, xt, Bmat, b_bil, gate_weight, blk):
    Rp, Cp = Dpad.shape
    ncols = H.shape[0]
    O = H.shape[1]
    grid = (Rp // blk,)
    body = functools.partial(_topk_bilinear_body, ncols)
    return pl.pallas_call(
        body,
        grid=grid,
        in_specs=[
            pl.BlockSpec((blk, Cp), lambda i: (i, 0)),
            pl.BlockSpec((Cp, O), lambda i: (0, 0)),
            pl.BlockSpec((blk, O), lambda i: (i, 0)),
            pl.BlockSpec((O * O, O), lambda i: (0, 0)),
            pl.BlockSpec((1, O), lambda i: (0, 0)),
            pl.BlockSpec((blk, O), lambda i: (i, 0)),
            pl.BlockSpec((blk, O), lambda i: (i, 0)),
            pl.BlockSpec((1, O), lambda i: (0, 0)),
        ],
        out_specs=pl.BlockSpec((blk, O), lambda i: (i, 0)),
        out_shape=jax.ShapeDtypeStruct((Rp, O), jnp.float32),
    )(Dpad, H, f, Bmat, b_bil.reshape(1, O), xh, xt,
      gate_weight.reshape(1, O))


def kernel(x, edge_index, W_homo, b_homo, W_hete, b_hete, B_bil, b_bil,
           gate_weight):
    N, D = x.shape
    O = W_homo.shape[0]
    S = S_SPLIT
    ND = N - S
    Sp = 896
    NDp = 9216
    row, col = edge_index[0], edge_index[1]

    deg = jnp.zeros((N,), x.dtype).at[col].add(1.0)
    dis = deg ** -0.5
    norm = dis[row] * dis[col]
    homo = ((row < S) & (col < S)) | ((row >= S) & (col >= S))
    w_homo_e = jnp.where(homo, norm, 0.0)
    w_hete_e = jnp.where(homo, 0.0, norm)
    x_homo = jnp.zeros_like(x).at[row].add(w_homo_e[:, None] * x[col])
    x_hete = jnp.zeros_like(x).at[row].add(w_hete_e[:, None] * x[col])

    m_mask = (row < S) & (col >= S)
    rm = jnp.where(m_mask, row, Sp - 1)
    cm = jnp.where(m_mask, col - S, NDp - 1)
    wm = jnp.where(m_mask, norm, 0.0)
    Dm = jnp.zeros((Sp, NDp), x.dtype).at[rm, cm].max(wm)

    d_mask = (row >= S) & (col < S)
    rd = jnp.where(d_mask, row - S, NDp - 1)
    cd = jnp.where(d_mask, col, Sp - 1)
    wd = jnp.where(d_mask, norm, 0.0)
    Dd = jnp.zeros((NDp, Sp), x.dtype).at[rd, cd].max(wd)

    x_homo_t, x_hete_t = _transforms(x_homo, x_hete, W_homo, b_homo,
                                     W_hete, b_hete)

    Bmat = B_bil.reshape(O, O * O).T.reshape(O * O, O)
    pad_m = jnp.zeros((Sp - S, O), jnp.float32)
    pad_d = jnp.zeros((NDp - ND, O), jnp.float32)
    H_dis = jnp.concatenate([x_hete_t[S:], pad_d], axis=0)
    H_mir = jnp.concatenate([x_hete_t[:S], pad_m], axis=0)
    f_m = H_mir
    xh_m = jnp.concatenate([x_homo_t[:S], pad_m], axis=0)
    out_m = _topk_bilinear(Dm, H_dis, f_m, xh_m, H_mir, Bmat, b_bil,
                           gate_weight, blk=64)
    f_d = H_dis
    xh_d = jnp.concatenate([x_homo_t[S:], pad_d], axis=0)
    out_d = _topk_bilinear(Dd, H_mir, f_d, xh_d, H_dis, Bmat, b_bil,
                           gate_weight, blk=128)
    return jnp.concatenate([out_m[:S], out_d[:ND]], axis=0)


# speedup vs baseline: 1.2293x; 1.2293x over previous
import functools
import jax
import jax.numpy as jnp
from jax import lax
from jax.experimental import pallas as pl
from jax.experimental.pallas import tpu as pltpu
from jax.experimental.pallas import tpu_sc as plsc

S_SPLIT = 812
K_TOP = 3
NEG_BIG = -1e30
_NC = 2
_NS = 16
_EC = 80


def _sc_scatter_feats(x, rowi, coli, w2, zeros_nd):
    N, D = x.shape
    E = rowi.shape[0]
    per_tile = E // _NS
    n_chunks = per_tile // _EC
    RB = 80
    n_rb = N // RB
    rb_rounds = -(-n_rb // _NS)
    mesh = plsc.VectorSubcoreMesh(core_axis_name="c", subcore_axis_name="s")

    @functools.partial(
        pl.kernel, mesh=mesh,
        out_type=jax.ShapeDtypeStruct((_NC, N, D), jnp.float32),
        scratch_types=[
            pltpu.VMEM((_EC,), jnp.int32),
            pltpu.VMEM((_EC,), jnp.int32),
            pltpu.VMEM((_EC + 16,), jnp.float32),
            pltpu.VMEM((_EC, D), jnp.float32),
            pltpu.VMEM_SHARED((N, D), jnp.float32),
            pltpu.SemaphoreType.DMA,
        ],
    )
    def k(x_hbm, row_hbm, col_hbm, w_hbm, z_hbm, out_hbm,
          ridx, cidx, wbuf, rows, acc, sem):
        cid = lax.axis_index("c")
        sid = lax.axis_index("s")
        for j in range(rb_rounds):
            bid = sid + j * _NS

            @pl.when(bid < n_rb)
            def _():
                pltpu.sync_copy(z_hbm.at[pl.ds(bid * RB, RB)],
                                acc.at[pl.ds(bid * RB, RB)])
        plsc.subcore_barrier()
        e0 = sid * per_tile

        def chunk_body(i, carry):
            off = e0 + i * _EC
            pltpu.sync_copy(row_hbm.at[pl.ds(off, _EC)], ridx)
            pltpu.sync_copy(col_hbm.at[pl.ds(off, _EC)], cidx)
            pltpu.sync_copy(w_hbm.at[pl.ds(cid * E + off, _EC)],
                            wbuf.at[pl.ds(0, _EC)])
            pltpu.async_copy(x_hbm.at[cidx], rows, sem).wait()

            def edge_body(e, c2):
                wv = jnp.full((16,), wbuf[pl.ds(e, 16)][0])
                for v in range(D // 16):
                    sl = pl.ds(v * 16, 16)
                    rows[e, sl] = rows[e, sl] * wv
                return c2

            lax.fori_loop(0, _EC, edge_body, 0)
            pltpu.sync_copy(rows, acc.at[ridx], add=True)
            return carry

        lax.fori_loop(0, n_chunks, chunk_body, 0)
        plsc.subcore_barrier()
        for j in range(rb_rounds):
            bid = sid + j * _NS

            @pl.when(bid < n_rb)
            def _():
                pltpu.sync_copy(acc.at[pl.ds(bid * RB, RB)],
                                out_hbm.at[cid, pl.ds(bid * RB, RB)])

    return k(x, rowi, coli, w2, zeros_nd)


def _transform_body(xh_ref, xt_ref, wh_ref, bh_ref, wt_ref, bt_ref,
                    oh_ref, ot_ref):
    xh = xh_ref[...]
    xt = xt_ref[...]
    oh_ref[...] = jnp.dot(xh, wh_ref[...],
                          preferred_element_type=jnp.float32) + bh_ref[...]
    ot_ref[...] = jnp.dot(xt, wt_ref[...],
                          preferred_element_type=jnp.float32) + bt_ref[...]


def _transforms(x_homo, x_hete, W_homo, b_homo, W_hete, b_hete):
    N, D = x_homo.shape
    O = W_homo.shape[0]
    BLK = 1000
    grid = (N // BLK,)
    whT = W_homo.T.reshape(D, O)
    wtT = W_hete.T.reshape(D, O)
    bh = b_homo.reshape(1, O)
    bt = b_hete.reshape(1, O)
    out = pl.pallas_call(
        _transform_body,
        grid=grid,
        in_specs=[
            pl.BlockSpec((BLK, D), lambda i: (i, 0)),
            pl.BlockSpec((BLK, D), lambda i: (i, 0)),
            pl.BlockSpec((D, O), lambda i: (0, 0)),
            pl.BlockSpec((1, O), lambda i: (0, 0)),
            pl.BlockSpec((D, O), lambda i: (0, 0)),
            pl.BlockSpec((1, O), lambda i: (0, 0)),
        ],
        out_specs=[
            pl.BlockSpec((BLK, O), lambda i: (i, 0)),
            pl.BlockSpec((BLK, O), lambda i: (i, 0)),
        ],
        out_shape=[
            jax.ShapeDtypeStruct((N, O), jnp.float32),
            jax.ShapeDtypeStruct((N, O), jnp.float32),
        ],
    )(x_homo, x_hete, whT, bh, wtT, bt)
    return out


def _topk_bilinear_body(ncols, d_ref, h_ref, f_ref, bmat_ref, bbil_ref,
                        xh_ref, xt_ref, gate_ref, out_ref):
    D = d_ref[...]
    R, C = D.shape
    col = lax.broadcasted_iota(jnp.int32, (R, C), 1)
    D = jnp.where(col < ncols, D, NEG_BIG)
    W = jnp.zeros((R, C), jnp.float32)
    cnt = jnp.zeros((R, 1), jnp.float32)
    for _ in range(K_TOP):
        m = jnp.max(D, axis=1, keepdims=True)
        is_max = D == m
        idx = jnp.min(jnp.where(is_max, col, jnp.int32(2**30)),
                      axis=1, keepdims=True)
        sel = col == idx
        valid = m > 0.0
        W = W + jnp.where(sel & valid, 1.0, 0.0)
        cnt = cnt + jnp.where(valid, 1.0, 0.0)
        D = jnp.where(sel, NEG_BIG, D)
    g = jnp.dot(W, h_ref[...], preferred_element_type=jnp.float32)
    g = g / jnp.maximum(cnt, 1.0)
    f = f_ref[...]
    O = f.shape[1]
    P = (f[:, :, None] * g[:, None, :]).reshape(R, O * O)
    mean = jnp.dot(P, bmat_ref[...],
                   preferred_element_type=jnp.float32) + bbil_ref[...]
    mean = jnp.where(cnt > 0.0, mean, 0.0)
    gate = jax.nn.sigmoid(gate_ref[...])
    leak = jnp.where(mean >= 0.0, mean, 0.01 * mean)
    fused = gate * leak + (1.0 - gate) * xt_ref[...]
    out_ref[...] = xh_ref[...] + fused


def _topk_bilinear(Dpad, H, f, xh, xt, Bmat, b_bil, gate_weight, blk):
    Rp, Cp = Dpad.shape
    ncols = H.shape[0]
    O = H.shape[1]
    grid = (Rp // blk,)
    body = functools.partial(_topk_bilinear_body, ncols)
    return pl.pallas_call(
        body,
        grid=grid,
        in_specs=[
            pl.BlockSpec((blk, Cp), lambda i: (i, 0)),
            pl.BlockSpec((Cp, O), lambda i: (0, 0)),
            pl.BlockSpec((blk, O), lambda i: (i, 0)),
            pl.BlockSpec((O * O, O), lambda i: (0, 0)),
            pl.BlockSpec((1, O), lambda i: (0, 0)),
            pl.BlockSpec((blk, O), lambda i: (i, 0)),
            pl.BlockSpec((blk, O), lambda i: (i, 0)),
            pl.BlockSpec((1, O), lambda i: (0, 0)),
        ],
        out_specs=pl.BlockSpec((blk, O), lambda i: (i, 0)),
        out_shape=jax.ShapeDtypeStruct((Rp, O), jnp.float32),
    )(Dpad, H, f, Bmat, b_bil.reshape(1, O), xh, xt,
      gate_weight.reshape(1, O))


def kernel(x, edge_index, W_homo, b_homo, W_hete, b_hete, B_bil, b_bil,
           gate_weight):
    N, D = x.shape
    O = W_homo.shape[0]
    S = S_SPLIT
    ND = N - S
    Sp = 896
    NDp = 9216
    row, col = edge_index[0], edge_index[1]

    deg = jnp.zeros((N,), x.dtype).at[col].add(1.0)
    dis = deg ** -0.5
    norm = dis[row] * dis[col]
    homo = ((row < S) & (col < S)) | ((row >= S) & (col >= S))
    w_homo_e = jnp.where(homo, norm, 0.0)
    w_hete_e = jnp.where(homo, 0.0, norm)
    w2 = jnp.concatenate([w_homo_e, w_hete_e])
    acc2 = _sc_scatter_feats(x, row, col, w2, jnp.zeros_like(x))
    x_homo = acc2[0]
    x_hete = acc2[1]

    m_mask = (row < S) & (col >= S)
    rm = jnp.where(m_mask, row, Sp - 1)
    cm = jnp.where(m_mask, col - S, NDp - 1)
    wm = jnp.where(m_mask, norm, 0.0)
    Dm = jnp.zeros((Sp, NDp), x.dtype).at[rm, cm].max(wm)

    d_mask = (row >= S) & (col < S)
    rd = jnp.where(d_mask, row - S, NDp - 1)
    cd = jnp.where(d_mask, col, Sp - 1)
    wd = jnp.where(d_mask, norm, 0.0)
    Dd = jnp.zeros((NDp, Sp), x.dtype).at[rd, cd].max(wd)

    x_homo_t, x_hete_t = _transforms(x_homo, x_hete, W_homo, b_homo,
                                     W_hete, b_hete)

    Bmat = B_bil.reshape(O, O * O).T.reshape(O * O, O)
    pad_m = jnp.zeros((Sp - S, O), jnp.float32)
    pad_d = jnp.zeros((NDp - ND, O), jnp.float32)
    H_dis = jnp.concatenate([x_hete_t[S:], pad_d], axis=0)
    H_mir = jnp.concatenate([x_hete_t[:S], pad_m], axis=0)
    f_m = H_mir
    xh_m = jnp.concatenate([x_homo_t[:S], pad_m], axis=0)
    out_m = _topk_bilinear(Dm, H_dis, f_m, xh_m, H_mir, Bmat, b_bil,
                           gate_weight, blk=64)
    f_d = H_dis
    xh_d = jnp.concatenate([x_homo_t[S:], pad_d], axis=0)
    out_d = _topk_bilinear(Dd, H_mir, f_d, xh_d, H_dis, Bmat, b_bil,
                           gate_weight, blk=128)
    return jnp.concatenate([out_m[:S], out_d[:ND]], axis=0)


# speedup vs baseline: 1.2676x; 1.0312x over previous
import functools
import jax
import jax.numpy as jnp
from jax import lax
from jax.experimental import pallas as pl
from jax.experimental.pallas import tpu as pltpu
from jax.experimental.pallas import tpu_sc as plsc

S_SPLIT = 812
K_TOP = 3
NEG_BIG = -1e30
_NC = 2
_NS = 16
_EC = 80


def _sc_scatter_feats(x, rowi, coli, w2, zeros_nd):
    N, D = x.shape
    E = rowi.shape[0]
    per_tile = E // _NS
    n_chunks = per_tile // _EC
    RB = 80
    n_rb = N // RB
    rb_rounds = -(-n_rb // _NS)
    mesh = plsc.VectorSubcoreMesh(core_axis_name="c", subcore_axis_name="s")

    @functools.partial(
        pl.kernel, mesh=mesh,
        out_type=jax.ShapeDtypeStruct((_NC, N, D), jnp.float32),
        scratch_types=[
            pltpu.VMEM((_EC,), jnp.int32),
            pltpu.VMEM((per_tile,), jnp.int32),
            pltpu.VMEM((per_tile + 16,), jnp.float32),
            pltpu.VMEM((_EC, D), jnp.float32),
            pltpu.VMEM_SHARED((N, D), jnp.float32),
            pltpu.SemaphoreType.DMA,
        ],
    )
    def k(x_hbm, row_hbm, col_hbm, w_hbm, z_hbm, out_hbm,
          ridx, cidx, wbuf, rows, acc, sem):
        cid = lax.axis_index("c")
        sid = lax.axis_index("s")
        e0 = sid * per_tile
        pltpu.sync_copy(col_hbm.at[pl.ds(e0, per_tile)], cidx)
        pltpu.sync_copy(w_hbm.at[pl.ds(cid * E + e0, per_tile)],
                        wbuf.at[pl.ds(0, per_tile)])
        for j in range(rb_rounds):
            bid = sid + j * _NS

            @pl.when(bid < n_rb)
            def _():
                pltpu.sync_copy(z_hbm.at[pl.ds(bid * RB, RB)],
                                acc.at[pl.ds(bid * RB, RB)])
        plsc.subcore_barrier()

        def chunk_body(i, carry):
            loc = i * _EC
            pltpu.sync_copy(row_hbm.at[pl.ds(e0 + loc, _EC)], ridx)
            pltpu.async_copy(x_hbm.at[cidx.at[pl.ds(loc, _EC)]],
                             rows, sem).wait()

            def edge_body(e, c2):
                wv = jnp.full((16,), wbuf[pl.ds(loc + e, 16)][0])
                for v in range(D // 16):
                    sl = pl.ds(v * 16, 16)
                    rows[e, sl] = rows[e, sl] * wv
                return c2

            lax.fori_loop(0, _EC, edge_body, 0)
            pltpu.sync_copy(rows, acc.at[ridx], add=True)
            return carry

        lax.fori_loop(0, n_chunks, chunk_body, 0)
        plsc.subcore_barrier()
        for j in range(rb_rounds):
            bid = sid + j * _NS

            @pl.when(bid < n_rb)
            def _():
                pltpu.sync_copy(acc.at[pl.ds(bid * RB, RB)],
                                out_hbm.at[cid, pl.ds(bid * RB, RB)])

    return k(x, rowi, coli, w2, zeros_nd)


def _transform_body(xh_ref, xt_ref, wh_ref, bh_ref, wt_ref, bt_ref,
                    oh_ref, ot_ref):
    xh = xh_ref[...]
    xt = xt_ref[...]
    oh_ref[...] = jnp.dot(xh, wh_ref[...],
                          preferred_element_type=jnp.float32) + bh_ref[...]
    ot_ref[...] = jnp.dot(xt, wt_ref[...],
                          preferred_element_type=jnp.float32) + bt_ref[...]


def _transforms(x_homo, x_hete, W_homo, b_homo, W_hete, b_hete):
    N, D = x_homo.shape
    O = W_homo.shape[0]
    BLK = 1000
    grid = (N // BLK,)
    whT = W_homo.T.reshape(D, O)
    wtT = W_hete.T.reshape(D, O)
    bh = b_homo.reshape(1, O)
    bt = b_hete.reshape(1, O)
    out = pl.pallas_call(
        _transform_body,
        grid=grid,
        in_specs=[
            pl.BlockSpec((BLK, D), lambda i: (i, 0)),
            pl.BlockSpec((BLK, D), lambda i: (i, 0)),
            pl.BlockSpec((D, O), lambda i: (0, 0)),
            pl.BlockSpec((1, O), lambda i: (0, 0)),
            pl.BlockSpec((D, O), lambda i: (0, 0)),
            pl.BlockSpec((1, O), lambda i: (0, 0)),
        ],
        out_specs=[
            pl.BlockSpec((BLK, O), lambda i: (i, 0)),
            pl.BlockSpec((BLK, O), lambda i: (i, 0)),
        ],
        out_shape=[
            jax.ShapeDtypeStruct((N, O), jnp.float32),
            jax.ShapeDtypeStruct((N, O), jnp.float32),
        ],
    )(x_homo, x_hete, whT, bh, wtT, bt)
    return out


def _topk_bilinear_body(ncols, d_ref, h_ref, f_ref, bmat_ref, bbil_ref,
                        xh_ref, xt_ref, gate_ref, out_ref):
    D = d_ref[...]
    R, C = D.shape
    col = lax.broadcasted_iota(jnp.int32, (R, C), 1)
    D = jnp.where(col < ncols, D, NEG_BIG)
    W = jnp.zeros((R, C), jnp.float32)
    cnt = jnp.zeros((R, 1), jnp.float32)
    for _ in range(K_TOP):
        m = jnp.max(D, axis=1, keepdims=True)
        is_max = D == m
        idx = jnp.min(jnp.where(is_max, col, jnp.int32(2**30)),
                      axis=1, keepdims=True)
        sel = col == idx
        valid = m > 0.0
        W = W + jnp.where(sel & valid, 1.0, 0.0)
        cnt = cnt + jnp.where(valid, 1.0, 0.0)
        D = jnp.where(sel, NEG_BIG, D)
    g = jnp.dot(W, h_ref[...], preferred_element_type=jnp.float32)
    g = g / jnp.maximum(cnt, 1.0)
    f = f_ref[...]
    O = f.shape[1]
    P = (f[:, :, None] * g[:, None, :]).reshape(R, O * O)
    mean = jnp.dot(P, bmat_ref[...],
                   preferred_element_type=jnp.float32) + bbil_ref[...]
    mean = jnp.where(cnt > 0.0, mean, 0.0)
    gate = jax.nn.sigmoid(gate_ref[...])
    leak = jnp.where(mean >= 0.0, mean, 0.01 * mean)
    fused = gate * leak + (1.0 - gate) * xt_ref[...]
    out_ref[...] = xh_ref[...] + fused


def _topk_bilinear(Dpad, H, f, xh, xt, Bmat, b_bil, gate_weight, blk):
    Rp, Cp = Dpad.shape
    ncols = H.shape[0]
    O = H.shape[1]
    grid = (Rp // blk,)
    body = functools.partial(_topk_bilinear_body, ncols)
    return pl.pallas_call(
        body,
        grid=grid,
        in_specs=[
            pl.BlockSpec((blk, Cp), lambda i: (i, 0)),
            pl.BlockSpec((Cp, O), lambda i: (0, 0)),
            pl.BlockSpec((blk, O), lambda i: (i, 0)),
            pl.BlockSpec((O * O, O), lambda i: (0, 0)),
            pl.BlockSpec((1, O), lambda i: (0, 0)),
            pl.BlockSpec((blk, O), lambda i: (i, 0)),
            pl.BlockSpec((blk, O), lambda i: (i, 0)),
            pl.BlockSpec((1, O), lambda i: (0, 0)),
        ],
        out_specs=pl.BlockSpec((blk, O), lambda i: (i, 0)),
        out_shape=jax.ShapeDtypeStruct((Rp, O), jnp.float32),
    )(Dpad, H, f, Bmat, b_bil.reshape(1, O), xh, xt,
      gate_weight.reshape(1, O))


def kernel(x, edge_index, W_homo, b_homo, W_hete, b_hete, B_bil, b_bil,
           gate_weight):
    N, D = x.shape
    O = W_homo.shape[0]
    S = S_SPLIT
    ND = N - S
    Sp = 896
    NDp = 9216
    row, col = edge_index[0], edge_index[1]

    deg = jnp.zeros((N,), x.dtype).at[col].add(1.0)
    dis = deg ** -0.5
    norm = dis[row] * dis[col]
    homo = ((row < S) & (col < S)) | ((row >= S) & (col >= S))
    w_homo_e = jnp.where(homo, norm, 0.0)
    w_hete_e = jnp.where(homo, 0.0, norm)
    w2 = jnp.concatenate([w_homo_e, w_hete_e])
    acc2 = _sc_scatter_feats(x, row, col, w2, jnp.zeros_like(x))
    x_homo = acc2[0]
    x_hete = acc2[1]

    m_mask = (row < S) & (col >= S)
    rm = jnp.where(m_mask, row, Sp - 1)
    cm = jnp.where(m_mask, col - S, NDp - 1)
    wm = jnp.where(m_mask, norm, 0.0)
    Dm = jnp.zeros((Sp, NDp), x.dtype).at[rm, cm].max(wm)

    d_mask = (row >= S) & (col < S)
    rd = jnp.where(d_mask, row - S, NDp - 1)
    cd = jnp.where(d_mask, col, Sp - 1)
    wd = jnp.where(d_mask, norm, 0.0)
    Dd = jnp.zeros((NDp, Sp), x.dtype).at[rd, cd].max(wd)

    x_homo_t, x_hete_t = _transforms(x_homo, x_hete, W_homo, b_homo,
                                     W_hete, b_hete)

    Bmat = B_bil.reshape(O, O * O).T.reshape(O * O, O)
    pad_m = jnp.zeros((Sp - S, O), jnp.float32)
    pad_d = jnp.zeros((NDp - ND, O), jnp.float32)
    H_dis = jnp.concatenate([x_hete_t[S:], pad_d], axis=0)
    H_mir = jnp.concatenate([x_hete_t[:S], pad_m], axis=0)
    f_m = H_mir
    xh_m = jnp.concatenate([x_homo_t[:S], pad_m], axis=0)
    out_m = _topk_bilinear(Dm, H_dis, f_m, xh_m, H_mir, Bmat, b_bil,
                           gate_weight, blk=64)
    f_d = H_dis
    xh_d = jnp.concatenate([x_homo_t[S:], pad_d], axis=0)
    out_d = _topk_bilinear(Dd, H_mir, f_d, xh_d, H_dis, Bmat, b_bil,
                           gate_weight, blk=128)
    return jnp.concatenate([out_m[:S], out_d[:ND]], axis=0)
